# baseline (speedup 1.0000x reference)
import jax
import jax.numpy as jnp
from jax.experimental import pallas as pl
from jax.experimental.pallas import tpu as pltpu

_H = 64
_NC = 8
_BF = jnp.bfloat16
_UNROLL = 128


def _cell(g, c_prev):
    s = 0.5 * jnp.tanh(g[:, : 3 * _H]) + 0.5
    gg = jnp.tanh(g[:, 3 * _H:])
    c = s[:, _H:2 * _H] * c_prev + s[:, : _H] * gg
    return (s[:, 2 * _H:] * jnp.tanh(c)).astype(_BF), c


def _step(x_t, h, c, w_ref, b_ref):
    g = jnp.dot(jnp.concatenate([x_t, h], axis=1), w_ref[...],
                preferred_element_type=jnp.float32) + b_ref[...]
    return _cell(g, c)


def _conv_bn_relu(x2, bk, w_ref, s_ref, t_ref):
    n = x2.shape[0]
    row = jax.lax.broadcasted_iota(jnp.int32, x2.shape, 0)
    xm = jnp.where(row >= bk, pltpu.roll(x2, bk, 0), 0.0)
    xp = jnp.where(row < n - bk, pltpu.roll(x2, n - bk, 0), 0.0)
    acc = jnp.dot(xm, w_ref[0], preferred_element_type=jnp.float32)
    acc = acc + jnp.dot(x2, w_ref[1], preferred_element_type=jnp.float32)
    acc = acc + jnp.dot(xp, w_ref[2], preferred_element_type=jnp.float32)
    return jnp.maximum(acc * s_ref[...] + t_ref[...], 0.0)


def _fused_kernel(
    x_ref,
    c1w, c1s, c1t,
    c2w, c2s, c2t,
    w0, w1,
    l0_b, l1_b,
    w2, l2_b,
    l3_wih, l3_b,
    head_w, head_b,
    o_ref,
    h2_ref,
    hf_ref, hr_ref,
):
    T, Bk, Cin = x_ref.shape
    n = T * Bk
    zero = jnp.zeros((Bk, _H), jnp.float32)
    zbf = jnp.zeros((Bk, _H), _BF)

    xb = x_ref[...].reshape(n, Cin)
    h1 = _conv_bn_relu(xb, Bk, c1w, c1s, c1t)
    h2_ref[...] = _conv_bn_relu(h1, Bk, c2w, c2s, c2t).astype(
        _BF).reshape(T, Bk, 2 * _H)

    def step1(i, carry):
        hf, cf, hr, cr = carry
        tr = T - 1 - i
        hf, cf = _step(h2_ref[i], hf, cf, w0, l0_b)
        hr, cr = _step(h2_ref[tr], hr, cr, w1, l1_b)
        hf_ref[i] = hf
        hr_ref[tr] = hr
        return hf, cf, hr, cr

    jax.lax.fori_loop(0, T, step1, (zbf, zero, zbf, zero), unroll=_UNROLL)

    def step2(i, carry):
        h, c = carry
        g = jnp.dot(jnp.concatenate([hf_ref[i], hr_ref[i], h], axis=1),
                    w2[...], preferred_element_type=jnp.float32) + l2_b[...]
        return _cell(g, c)

    h2f, _ = jax.lax.fori_loop(0, T, step2, (zbf, zero), unroll=_UNROLL)

    g_rev = (jnp.dot(jnp.concatenate([hf_ref[T - 1], hr_ref[T - 1]], axis=1),
                     l3_wih[...], preferred_element_type=jnp.float32)
             + l3_b[...])
    h2r, _ = _cell(g_rev, zero)

    relu2 = jnp.concatenate(
        [jnp.maximum(h2f.astype(jnp.float32), 0.0),
         jnp.maximum(h2r.astype(jnp.float32), 0.0)], axis=1).astype(_BF)
    o_ref[...] = (jnp.dot(relu2, head_w[...],
                          preferred_element_type=jnp.float32) + head_b[...])


def kernel(x, c1w, c1s, c1t, c2w, c2s, c2t,
           l0_wih, l0_whh, l0_b, l1_wih, l1_whh, l1_b,
           l2_wih, l2_whh, l2_b, l3_wih, l3_whh, l3_b,
           head_w, head_b):
    B, Cin, T = x.shape
    xt = jnp.transpose(x, (2, 0, 1))

    Bk = 128
    while B % Bk:
        Bk //= 2

    gsc = jnp.concatenate(
        [jnp.full((1, 3 * _H), 0.5, jnp.float32),
         jnp.ones((1, _H), jnp.float32)], axis=1)
    w0 = (jnp.concatenate([l0_wih, l0_whh], axis=0) * gsc).astype(_BF)
    w1 = (jnp.concatenate([l1_wih, l1_whh], axis=0) * gsc).astype(_BF)
    w2 = (jnp.concatenate([l2_wih, l2_whh], axis=0) * gsc).astype(_BF)
    l3w = (l3_wih * gsc).astype(_BF)
    l0b, l1b, l2b, l3b = (b * gsc for b in (l0_b, l1_b, l2_b, l3_b))

    full = lambda *shape: pl.BlockSpec(shape, lambda b: (0,) * len(shape))
    out = pl.pallas_call(
        _fused_kernel,
        out_shape=jax.ShapeDtypeStruct((B, _NC), jnp.float32),
        grid=(B // Bk,),
        in_specs=[
            pl.BlockSpec((T, Bk, Cin), lambda b: (0, b, 0)),
            full(3, Cin, 64), full(1, 64), full(1, 64),
            full(3, 64, 128), full(1, 128), full(1, 128),
            full(3 * _H, 4 * _H), full(3 * _H, 4 * _H),
            full(1, 4 * _H), full(1, 4 * _H),
            full(3 * _H, 4 * _H), full(1, 4 * _H),
            full(2 * _H, 4 * _H), full(1, 4 * _H),
            full(2 * _H, _NC), full(1, _NC),
        ],
        out_specs=pl.BlockSpec((Bk, _NC), lambda b: (b, 0)),
        scratch_shapes=[
            pltpu.VMEM((T, Bk, 2 * _H), _BF),
            pltpu.VMEM((T, Bk, _H), _BF),
            pltpu.VMEM((T, Bk, _H), _BF),
        ],
        compiler_params=pltpu.CompilerParams(
            dimension_semantics=("parallel",)),
    )(
        xt, c1w, c1s, c1t, c2w, c2s, c2t,
        w0, w1, l0b, l1b, w2, l2b, l3w, l3b,
        head_w.astype(_BF), head_b,
    )
    return out


# speedup vs baseline: 1.0959x; 1.0959x over previous
import jax
import jax.numpy as jnp
from jax.experimental import pallas as pl
from jax.experimental.pallas import tpu as pltpu

_H = 64
_NC = 8
_BF = jnp.bfloat16
_UNROLL = 64


def _cell(g, c_prev):
    s = 0.5 * jnp.tanh(g[:, : 3 * _H]) + 0.5
    gg = jnp.tanh(g[:, 3 * _H:])
    c = s[:, _H:2 * _H] * c_prev + s[:, : _H] * gg
    return (s[:, 2 * _H:] * jnp.tanh(c)).astype(_BF), c


def _step(x_t, h, c, w_ref, b_ref):
    g = jnp.dot(jnp.concatenate([x_t, h], axis=1), w_ref[...],
                preferred_element_type=jnp.float32) + b_ref[...]
    return _cell(g, c)


def _conv_bn_relu(x2, bk, w_ref, s_ref, t_ref):
    n = x2.shape[0]
    row = jax.lax.broadcasted_iota(jnp.int32, x2.shape, 0)
    xm = jnp.where(row >= bk, pltpu.roll(x2, bk, 0), 0.0)
    xp = jnp.where(row < n - bk, pltpu.roll(x2, n - bk, 0), 0.0)
    acc = jnp.dot(xm, w_ref[0], preferred_element_type=jnp.float32)
    acc = acc + jnp.dot(x2, w_ref[1], preferred_element_type=jnp.float32)
    acc = acc + jnp.dot(xp, w_ref[2], preferred_element_type=jnp.float32)
    return jnp.maximum(acc * s_ref[...] + t_ref[...], 0.0)


def _fused_kernel(
    x_ref,
    c1w, c1s, c1t,
    c2w, c2s, c2t,
    w0, w1,
    l0_b, l1_b,
    w2, l2_b,
    l3_wih, l3_b,
    head_w, head_b,
    o_ref,
    h2_ref,
    hfr_ref,
):
    T, Bk, Cin = x_ref.shape
    n = T * Bk
    zero = jnp.zeros((Bk, _H), jnp.float32)
    zbf = jnp.zeros((Bk, _H), _BF)

    xb = x_ref[...].reshape(n, Cin)
    h1 = _conv_bn_relu(xb, Bk, c1w, c1s, c1t)
    h2_ref[...] = _conv_bn_relu(h1, Bk, c2w, c2s, c2t).astype(
        _BF).reshape(T, Bk, 2 * _H)

    def step1(i, carry):
        hf, cf, hr, cr = carry
        tr = T - 1 - i
        hf, cf = _step(h2_ref[i], hf, cf, w0, l0_b)
        hr, cr = _step(h2_ref[tr], hr, cr, w1, l1_b)
        hfr_ref[i, :, : _H] = hf
        hfr_ref[tr, :, _H:] = hr
        return hf, cf, hr, cr

    jax.lax.fori_loop(0, T, step1, (zbf, zero, zbf, zero), unroll=_UNROLL)

    def step2(i, carry):
        h, c = carry
        g = jnp.dot(jnp.concatenate([hfr_ref[i], h], axis=1),
                    w2[...], preferred_element_type=jnp.float32) + l2_b[...]
        return _cell(g, c)

    h2f, _ = jax.lax.fori_loop(0, T, step2, (zbf, zero), unroll=_UNROLL)

    g_rev = (jnp.dot(hfr_ref[T - 1], l3_wih[...],
                     preferred_element_type=jnp.float32)
             + l3_b[...])
    h2r, _ = _cell(g_rev, zero)

    relu2 = jnp.concatenate(
        [jnp.maximum(h2f.astype(jnp.float32), 0.0),
         jnp.maximum(h2r.astype(jnp.float32), 0.0)], axis=1).astype(_BF)
    o_ref[...] = (jnp.dot(relu2, head_w[...],
                          preferred_element_type=jnp.float32) + head_b[...])


def kernel(x, c1w, c1s, c1t, c2w, c2s, c2t,
           l0_wih, l0_whh, l0_b, l1_wih, l1_whh, l1_b,
           l2_wih, l2_whh, l2_b, l3_wih, l3_whh, l3_b,
           head_w, head_b):
    B, Cin, T = x.shape
    xt = jnp.transpose(x, (2, 0, 1))

    Bk = 128
    while B % Bk:
        Bk //= 2

    gsc = jnp.concatenate(
        [jnp.full((1, 3 * _H), 0.5, jnp.float32),
         jnp.ones((1, _H), jnp.float32)], axis=1)
    w0 = (jnp.concatenate([l0_wih, l0_whh], axis=0) * gsc).astype(_BF)
    w1 = (jnp.concatenate([l1_wih, l1_whh], axis=0) * gsc).astype(_BF)
    w2 = (jnp.concatenate([l2_wih, l2_whh], axis=0) * gsc).astype(_BF)
    l3w = (l3_wih * gsc).astype(_BF)
    l0b, l1b, l2b, l3b = (b * gsc for b in (l0_b, l1_b, l2_b, l3_b))

    full = lambda *shape: pl.BlockSpec(shape, lambda b: (0,) * len(shape))
    out = pl.pallas_call(
        _fused_kernel,
        out_shape=jax.ShapeDtypeStruct((B, _NC), jnp.float32),
        grid=(B // Bk,),
        in_specs=[
            pl.BlockSpec((T, Bk, Cin), lambda b: (0, b, 0)),
            full(3, Cin, 64), full(1, 64), full(1, 64),
            full(3, 64, 128), full(1, 128), full(1, 128),
            full(3 * _H, 4 * _H), full(3 * _H, 4 * _H),
            full(1, 4 * _H), full(1, 4 * _H),
            full(3 * _H, 4 * _H), full(1, 4 * _H),
            full(2 * _H, 4 * _H), full(1, 4 * _H),
            full(2 * _H, _NC), full(1, _NC),
        ],
        out_specs=pl.BlockSpec((Bk, _NC), lambda b: (b, 0)),
        scratch_shapes=[
            pltpu.VMEM((T, Bk, 2 * _H), _BF),
            pltpu.VMEM((T, Bk, 2 * _H), _BF),
        ],
        compiler_params=pltpu.CompilerParams(
            dimension_semantics=("parallel",)),
    )(
        xt, c1w, c1s, c1t, c2w, c2s, c2t,
        w0, w1, l0b, l1b, w2, l2b, l3w, l3b,
        head_w.astype(_BF), head_b,
    )
    return out
